# Initial kernel scaffold; baseline (speedup 1.0000x reference)
import jax
import jax.numpy as jnp
from jax import lax
from jax.experimental import pallas as pl
from jax.experimental.pallas import tpu as pltpu

_T = 4096
_L = 64
_D = 256


def _crf_body(feats_ref, w_ref, b_ref, start_ref, trans_ref, end_ref,
              score_ref, path_ref, em_ref, scores_ref):
    em_ref[...] = (
        jnp.dot(feats_ref[...], w_ref[...],
                preferred_element_type=jnp.float32)
        + b_ref[...]
    )

    lane = lax.broadcasted_iota(jnp.int32, (1, _L), 1)

    s0 = start_ref[...] + em_ref[0:1, :]
    scores_ref[0:1, :] = s0

    def fwd(t, s):
        swt = s + trans_ref[...]
        mx = jnp.max(swt, axis=1)
        s_new = em_ref[pl.ds(t, 1), :] + mx[None, :]
        scores_ref[pl.ds(t, 1), :] = s_new
        return s_new

    s_fin = lax.fori_loop(1, _T, fwd, s0)

    final = s_fin + end_ref[...]
    vs = jnp.max(final)
    score_ref[0, 0] = vs
    last = jnp.min(jnp.where(final == vs, lane, _L)).astype(jnp.int32)
    path_ref[_T - 1:_T, :] = jnp.full((1, 1), last, jnp.int32)

    def bwd(i, c):
        k = _T - 2 - i
        v = scores_ref[pl.ds(k, 1), :] + trans_ref[pl.ds(c, 1), :]
        m = jnp.max(v)
        prev = jnp.min(jnp.where(v == m, lane, _L)).astype(jnp.int32)
        path_ref[pl.ds(k, 1), :] = jnp.full((1, 1), prev, jnp.int32)
        return prev

    lax.fori_loop(0, _T - 1, bwd, last)


def kernel(feats, batch_sizes, W, b, start_transition, transitions,
           end_transition):
    del batch_sizes
    score, path = pl.pallas_call(
        _crf_body,
        out_shape=[
            jax.ShapeDtypeStruct((1, 1), jnp.float32),
            jax.ShapeDtypeStruct((_T, 1), jnp.int32),
        ],
        scratch_shapes=[
            pltpu.VMEM((_T, _L), jnp.float32),
            pltpu.VMEM((_T, _L), jnp.float32),
        ],
    )(
        feats,
        W,
        b.reshape(1, _L),
        start_transition.reshape(1, _L),
        transitions,
        end_transition.reshape(1, _L),
    )
    return score[0, 0], path[:, 0]


# speedup vs baseline: 11.6110x; 11.6110x over previous
import jax
import jax.numpy as jnp
from jax import lax
from jax.experimental import pallas as pl
from jax.experimental.pallas import tpu as pltpu

_T = 4096
_L = 64
_D = 256


def _crf_body(feats_ref, w_ref, b_ref, start_ref, trans_ref, end_ref,
              score_ref, path_ref, em_ref, scores_ref):
    em_ref[...] = (
        jnp.dot(feats_ref[...], w_ref[...],
                preferred_element_type=jnp.float32)
        + b_ref[...]
    )

    lane = lax.broadcasted_iota(jnp.int32, (1, _L), 1)

    s0 = start_ref[...] + em_ref[0:1, :]
    scores_ref[0:1, :] = s0

    def fwd(t, s):
        swt = s + trans_ref[...]
        mx = jnp.max(swt, axis=1)
        s_new = em_ref[pl.ds(t, 1), :] + mx[None, :]
        scores_ref[pl.ds(t, 1), :] = s_new
        return s_new

    s_fin = lax.fori_loop(1, _T, fwd, s0)

    final = s_fin + end_ref[...]
    vs = jnp.max(final)
    score_ref[...] = jnp.full((1, 1), vs, jnp.float32)
    last = jnp.min(jnp.where(final == vs, lane, _L)).astype(jnp.int32)
    path_ref[_T - 1:_T, :] = jnp.full((1, 1), last, jnp.int32)

    def bwd(i, c):
        k = _T - 2 - i
        v = scores_ref[pl.ds(k, 1), :] + trans_ref[pl.ds(c, 1), :]
        m = jnp.max(v)
        prev = jnp.min(jnp.where(v == m, lane, _L)).astype(jnp.int32)
        path_ref[pl.ds(k, 1), :] = jnp.full((1, 1), prev, jnp.int32)
        return prev

    lax.fori_loop(0, _T - 1, bwd, last)


def kernel(feats, batch_sizes, W, b, start_transition, transitions,
           end_transition):
    del batch_sizes
    score, path = pl.pallas_call(
        _crf_body,
        out_shape=[
            jax.ShapeDtypeStruct((1, 1), jnp.float32),
            jax.ShapeDtypeStruct((_T, 1), jnp.int32),
        ],
        scratch_shapes=[
            pltpu.VMEM((_T, _L), jnp.float32),
            pltpu.VMEM((_T, _L), jnp.float32),
        ],
    )(
        feats,
        W,
        b.reshape(1, _L),
        start_transition.reshape(1, _L),
        transitions,
        end_transition.reshape(1, _L),
    )
    return score[0, 0], path[:, 0]


# speedup vs baseline: 20.1987x; 1.7396x over previous
import jax
import jax.numpy as jnp
from jax import lax
from jax.experimental import pallas as pl
from jax.experimental.pallas import tpu as pltpu

_T = 4096
_L = 64
_D = 256
_H = _T // 2


def _crf_body(feats_ref, w_ref, b_row_ref, start_ref,
              t_ref, tt_ref, end_col_ref,
              score_ref, path_ref, em_ref, bps_ref):
    f32 = jnp.float32

    em_ref[...] = (
        jnp.dot(feats_ref[...], w_ref[...], preferred_element_type=f32)
        + b_row_ref[...]
    )

    liota_m = lax.broadcasted_iota(jnp.int32, (_L, _L), 1).astype(f32)
    siota_m = lax.broadcasted_iota(jnp.int32, (_L, _L), 0).astype(f32)
    liota_r = lax.broadcasted_iota(jnp.int32, (1, _L), 1).astype(f32)
    siota_c = lax.broadcasted_iota(jnp.int32, (_L, 1), 0).astype(f32)

    tmat = t_ref[...]
    ttmat = tt_ref[...]

    s0 = start_ref[...] + em_ref[0:1, :]

    def fwd_pair(it, s_row):
        swt = s_row + tmat
        mxa = jnp.max(swt, axis=1, keepdims=True)
        bpa = jnp.min(jnp.where(swt == mxa, liota_m, float(_L)),
                      axis=1, keepdims=True)
        bps_ref[pl.ds(2 * it, 1), :] = bpa.reshape(1, _L)
        em_col = em_ref[pl.ds(2 * it + 1, 1), :].reshape(_L, 1)
        s_col = em_col + mxa
        swt2 = s_col + ttmat
        mxb = jnp.max(swt2, axis=0, keepdims=True)
        bpb = jnp.min(jnp.where(swt2 == mxb, siota_m, float(_L)),
                      axis=0, keepdims=True)
        bps_ref[pl.ds(2 * it + 1, 1), :] = bpb
        return em_ref[pl.ds(2 * it + 2, 1), :] + mxb

    s_row = lax.fori_loop(0, _H - 1, fwd_pair, s0)

    swt = s_row + tmat
    mxa = jnp.max(swt, axis=1, keepdims=True)
    bpa = jnp.min(jnp.where(swt == mxa, liota_m, float(_L)),
                  axis=1, keepdims=True)
    bps_ref[_T - 2:_T - 1, :] = bpa.reshape(1, _L)
    em_col = em_ref[_T - 1:_T, :].reshape(_L, 1)
    final = em_col + mxa + end_col_ref[...]
    vs = jnp.max(final)
    score_ref[...] = jnp.full((1, 1), vs, f32)
    last = jnp.min(jnp.where(final == vs, siota_c, float(_L)),
                   axis=0, keepdims=True)
    path_ref[_T - 1:_T, :] = last.astype(jnp.int32)

    def bwd(i, c):
        k = _T - 2 - i
        row = bps_ref[pl.ds(k, 1), :]
        c = jnp.max(jnp.where(liota_r == c, row, -1.0),
                    axis=1, keepdims=True)
        path_ref[pl.ds(k, 1), :] = c.astype(jnp.int32)
        return c

    lax.fori_loop(0, _T - 1, bwd, last)


def kernel(feats, batch_sizes, W, b, start_transition, transitions,
           end_transition):
    del batch_sizes
    score, path = pl.pallas_call(
        _crf_body,
        out_shape=[
            jax.ShapeDtypeStruct((1, 1), jnp.float32),
            jax.ShapeDtypeStruct((_T, 1), jnp.int32),
        ],
        scratch_shapes=[
            pltpu.VMEM((_T, _L), jnp.float32),
            pltpu.VMEM((_T, _L), jnp.float32),
        ],
    )(
        feats,
        W,
        b.reshape(1, _L),
        start_transition.reshape(1, _L),
        transitions,
        transitions.T,
        end_transition.reshape(_L, 1),
    )
    return score[0, 0], path[:, 0]


# speedup vs baseline: 52.5578x; 2.6020x over previous
import jax
import jax.numpy as jnp
from jax import lax
from jax.experimental import pallas as pl
from jax.experimental.pallas import tpu as pltpu

_T = 4096
_L = 64
_D = 256
_H = _T // 2


def _compose(src, idx):
    return jnp.take_along_axis(src, idx, axis=1)


def _crf_body(feats_ref, w_ref, b_row_ref, start_ref,
              t_ref, tt_ref, end_col_ref,
              score_ref, pev_ref, pod_ref,
              em_ref, bpsa_ref, bpsb_ref, h0_ref, h1_ref):
    f32 = jnp.float32
    i32 = jnp.int32

    em_ref[...] = (
        jnp.dot(feats_ref[...], w_ref[...], preferred_element_type=f32)
        + b_row_ref[...]
    )

    liota_m = lax.broadcasted_iota(i32, (_L, _L), 1).astype(f32)
    siota_m = lax.broadcasted_iota(i32, (_L, _L), 0).astype(f32)
    siota_c = lax.broadcasted_iota(i32, (_L, 1), 0).astype(f32)

    tmat = t_ref[...]
    ttmat = tt_ref[...]

    s0 = start_ref[...] + em_ref[0:1, :]

    def fwd_pair(it, s_row):
        swt = s_row + tmat
        mxa = jnp.max(swt, axis=1, keepdims=True)
        bpa = jnp.min(jnp.where(swt == mxa, liota_m, float(_L)),
                      axis=1, keepdims=True)
        bpsa_ref[pl.ds(it, 1), :] = bpa.reshape(1, _L).astype(i32)
        em_col = em_ref[pl.ds(2 * it + 1, 1), :].reshape(_L, 1)
        s_col = em_col + mxa
        swt2 = s_col + ttmat
        mxb = jnp.max(swt2, axis=0, keepdims=True)
        bpb = jnp.min(jnp.where(swt2 == mxb, siota_m, float(_L)),
                      axis=0, keepdims=True)
        bpsb_ref[pl.ds(it, 1), :] = bpb.astype(i32)
        return em_ref[pl.ds(2 * it + 2, 1), :] + mxb

    s_row = lax.fori_loop(0, _H - 1, fwd_pair, s0)

    swt = s_row + tmat
    mxa = jnp.max(swt, axis=1, keepdims=True)
    bpa = jnp.min(jnp.where(swt == mxa, liota_m, float(_L)),
                  axis=1, keepdims=True)
    bpsa_ref[_H - 1:_H, :] = bpa.reshape(1, _L).astype(i32)
    bpsb_ref[_H - 1:_H, :] = lax.broadcasted_iota(i32, (1, _L), 1)
    em_col = em_ref[_T - 1:_T, :].reshape(_L, 1)
    final = em_col + mxa + end_col_ref[...]
    vs = jnp.max(final)
    score_ref[...] = jnp.full((1, 1), vs, f32)
    last = jnp.min(jnp.where(final == vs, siota_c, float(_L)),
                   axis=0, keepdims=True).astype(i32)

    h0_ref[...] = _compose(bpsa_ref[...], bpsb_ref[...])

    src, dst = h0_ref, h1_ref
    off = 1
    while off < _H:
        n = _H - off
        dst[0:n, :] = _compose(src[0:n, :], src[off:_H, :])
        dst[n:_H, :] = src[n:_H, :]
        src, dst = dst, src
        off *= 2

    hodd = _compose(
        bpsb_ref[...],
        jnp.concatenate([src[1:_H, :],
                         lax.broadcasted_iota(i32, (1, _L), 1)], axis=0))

    idx_ev = jnp.broadcast_to(last, (_H, 1))
    pev_ref[...] = jnp.take_along_axis(src[...], idx_ev, axis=1)
    pod_ref[...] = jnp.take_along_axis(hodd, idx_ev, axis=1)


def kernel(feats, batch_sizes, W, b, start_transition, transitions,
           end_transition):
    del batch_sizes
    score, pev, pod = pl.pallas_call(
        _crf_body,
        out_shape=[
            jax.ShapeDtypeStruct((1, 1), jnp.float32),
            jax.ShapeDtypeStruct((_H, 1), jnp.int32),
            jax.ShapeDtypeStruct((_H, 1), jnp.int32),
        ],
        scratch_shapes=[
            pltpu.VMEM((_T, _L), jnp.float32),
            pltpu.VMEM((_H, _L), jnp.int32),
            pltpu.VMEM((_H, _L), jnp.int32),
            pltpu.VMEM((_H, _L), jnp.int32),
            pltpu.VMEM((_H, _L), jnp.int32),
        ],
    )(
        feats,
        W,
        b.reshape(1, _L),
        start_transition.reshape(1, _L),
        transitions,
        transitions.T,
        end_transition.reshape(_L, 1),
    )
    path = jnp.stack([pev[:, 0], pod[:, 0]], axis=1).reshape(_T)
    return score[0, 0], path
